# baseline (device time: 77387 ns/iter reference)
import jax
import jax.numpy as jnp
from jax import lax
from jax.experimental import pallas as pl
from jax.experimental.pallas import tpu as pltpu

N_DEV = 16


def kernel(x, W1, W2):
    m, _ = x.shape
    n = W2.shape[1]

    def body(x_ref, w1_ref, w2_ref, out_ref, comm_ref, send_sems, recv_sems):
        my = lax.axis_index("i")
        left = (my - 1) % N_DEV
        right = (my + 1) % N_DEV

        barrier_sem = pltpu.get_barrier_semaphore()
        for nbr in (left, right):
            pl.semaphore_signal(
                barrier_sem, inc=1,
                device_id=(nbr,), device_id_type=pl.DeviceIdType.MESH,
            )
        pl.semaphore_wait(barrier_sem, 2)

        hidden = jnp.maximum(
            jnp.dot(x_ref[...], w1_ref[...], preferred_element_type=jnp.float32),
            0.0,
        )
        partial = jnp.dot(hidden, w2_ref[...], preferred_element_type=jnp.float32)
        comm_ref[0] = partial

        acc = partial
        for hop in range(N_DEV - 1):
            rdma = pltpu.make_async_remote_copy(
                src_ref=comm_ref.at[hop],
                dst_ref=comm_ref.at[hop + 1],
                send_sem=send_sems.at[hop],
                recv_sem=recv_sems.at[hop],
                device_id=(right,),
                device_id_type=pl.DeviceIdType.MESH,
            )
            rdma.start()
            rdma.wait()
            acc = acc + comm_ref[hop + 1]
        out_ref[...] = acc

    return pl.pallas_call(
        body,
        out_shape=jax.ShapeDtypeStruct((m, n), jnp.float32),
        in_specs=[
            pl.BlockSpec(memory_space=pltpu.VMEM),
            pl.BlockSpec(memory_space=pltpu.VMEM),
            pl.BlockSpec(memory_space=pltpu.VMEM),
        ],
        out_specs=pl.BlockSpec(memory_space=pltpu.VMEM),
        scratch_shapes=[
            pltpu.VMEM((N_DEV, m, n), jnp.float32),
            pltpu.SemaphoreType.DMA((N_DEV - 1,)),
            pltpu.SemaphoreType.DMA((N_DEV - 1,)),
        ],
        compiler_params=pltpu.CompilerParams(collective_id=0),
    )(x, W1, W2)


# device time: 30586 ns/iter; 2.5301x vs baseline; 2.5301x over previous
import jax
import jax.numpy as jnp
from jax import lax
from jax.experimental import pallas as pl
from jax.experimental.pallas import tpu as pltpu

N_DEV = 16


def kernel(x, W1, W2):
    m, _ = x.shape
    n = W2.shape[1]

    STEPS = 4

    def body(x_ref, w1_ref, w2_ref, out_ref, comm_ref, send_sems, recv_sems):
        my = lax.axis_index("i")
        partners = [my ^ (1 << s) for s in range(STEPS)]

        barrier_sem = pltpu.get_barrier_semaphore()
        for p in partners:
            pl.semaphore_signal(
                barrier_sem, inc=1,
                device_id=(p,), device_id_type=pl.DeviceIdType.MESH,
            )
        pl.semaphore_wait(barrier_sem, STEPS)

        hidden = jnp.maximum(
            jnp.dot(x_ref[...], w1_ref[...], preferred_element_type=jnp.float32),
            0.0,
        )
        acc = jnp.dot(hidden, w2_ref[...], preferred_element_type=jnp.float32)

        for s in range(STEPS):
            comm_ref[2 * s] = acc
            rdma = pltpu.make_async_remote_copy(
                src_ref=comm_ref.at[2 * s],
                dst_ref=comm_ref.at[2 * s + 1],
                send_sem=send_sems.at[s],
                recv_sem=recv_sems.at[s],
                device_id=(partners[s],),
                device_id_type=pl.DeviceIdType.MESH,
            )
            rdma.start()
            rdma.wait()
            acc = acc + comm_ref[2 * s + 1]
        out_ref[...] = acc

    return pl.pallas_call(
        body,
        out_shape=jax.ShapeDtypeStruct((m, n), jnp.float32),
        in_specs=[
            pl.BlockSpec(memory_space=pltpu.VMEM),
            pl.BlockSpec(memory_space=pltpu.VMEM),
            pl.BlockSpec(memory_space=pltpu.VMEM),
        ],
        out_specs=pl.BlockSpec(memory_space=pltpu.VMEM),
        scratch_shapes=[
            pltpu.VMEM((2 * STEPS, m, n), jnp.float32),
            pltpu.SemaphoreType.DMA((STEPS,)),
            pltpu.SemaphoreType.DMA((STEPS,)),
        ],
        compiler_params=pltpu.CompilerParams(collective_id=0),
    )(x, W1, W2)


# device time: 22872 ns/iter; 3.3835x vs baseline; 1.3373x over previous
import jax
import jax.numpy as jnp
from jax import lax
from jax.experimental import pallas as pl
from jax.experimental.pallas import tpu as pltpu

N_DEV = 16


def kernel(x, W1, W2):
    m, _ = x.shape
    n = W2.shape[1]

    STEPS = 4
    NCHUNKS = 4
    mc = m // NCHUNKS

    def body(x_ref, w1_ref, w2_ref, out_ref, comm_ref, send_sems, recv_sems):
        my = lax.axis_index("i")
        partner = [
            [my ^ (1 << ((s + c) % STEPS)) for s in range(STEPS)]
            for c in range(NCHUNKS)
        ]
        partners_all = [my ^ (1 << b) for b in range(STEPS)]

        barrier_sem = pltpu.get_barrier_semaphore()
        for p in partners_all:
            pl.semaphore_signal(
                barrier_sem, inc=1,
                device_id=(p,), device_id_type=pl.DeviceIdType.MESH,
            )
        pl.semaphore_wait(barrier_sem, STEPS)

        hidden = jnp.maximum(
            jnp.dot(x_ref[...], w1_ref[...], preferred_element_type=jnp.float32),
            0.0,
        )
        partial = jnp.dot(hidden, w2_ref[...], preferred_element_type=jnp.float32)
        acc = [partial[c * mc:(c + 1) * mc, :] for c in range(NCHUNKS)]

        def make_rdma(c, s):
            return pltpu.make_async_remote_copy(
                src_ref=comm_ref.at[c, 2 * s],
                dst_ref=comm_ref.at[c, 2 * s + 1],
                send_sem=send_sems.at[c, s],
                recv_sem=recv_sems.at[c, s],
                device_id=(partner[c][s],),
                device_id_type=pl.DeviceIdType.MESH,
            )

        rdmas = [None] * NCHUNKS
        for c in range(NCHUNKS):
            comm_ref[c, 0] = acc[c]
            rdmas[c] = make_rdma(c, 0)
            rdmas[c].start()
        for s in range(STEPS):
            for c in range(NCHUNKS):
                rdmas[c].wait()
                acc[c] = acc[c] + comm_ref[c, 2 * s + 1]
                if s + 1 < STEPS:
                    comm_ref[c, 2 * (s + 1)] = acc[c]
                    rdmas[c] = make_rdma(c, s + 1)
                    rdmas[c].start()
        for c in range(NCHUNKS):
            out_ref[c * mc:(c + 1) * mc, :] = acc[c]

    return pl.pallas_call(
        body,
        out_shape=jax.ShapeDtypeStruct((m, n), jnp.float32),
        in_specs=[
            pl.BlockSpec(memory_space=pltpu.VMEM),
            pl.BlockSpec(memory_space=pltpu.VMEM),
            pl.BlockSpec(memory_space=pltpu.VMEM),
        ],
        out_specs=pl.BlockSpec(memory_space=pltpu.VMEM),
        scratch_shapes=[
            pltpu.VMEM((NCHUNKS, 2 * STEPS, mc, n), jnp.float32),
            pltpu.SemaphoreType.DMA((NCHUNKS, STEPS)),
            pltpu.SemaphoreType.DMA((NCHUNKS, STEPS)),
        ],
        compiler_params=pltpu.CompilerParams(collective_id=0),
    )(x, W1, W2)


# device time: 19231 ns/iter; 4.0241x vs baseline; 1.1893x over previous
import jax
import jax.numpy as jnp
from jax import lax
from jax.experimental import pallas as pl
from jax.experimental.pallas import tpu as pltpu

N_DEV = 16


def kernel(x, W1, W2):
    m, _ = x.shape
    n = W2.shape[1]

    STEPS = 4
    NCHUNKS = 4
    mc = m // NCHUNKS

    def body(x_ref, w1_ref, w2_ref, out_ref, comm_ref, send_sems, recv_sems):
        my = lax.axis_index("i")
        p_in_plane = my & 3
        dim_partner = [
            my ^ 1,
            my + 3 - 2 * p_in_plane,
            my ^ 4,
            my ^ 8,
        ]
        partner = [
            [dim_partner[(s + c) % STEPS] for s in range(STEPS)]
            for c in range(NCHUNKS)
        ]
        partners_all = dim_partner

        barrier_sem = pltpu.get_barrier_semaphore()
        for p in partners_all:
            pl.semaphore_signal(
                barrier_sem, inc=1,
                device_id=(p,), device_id_type=pl.DeviceIdType.MESH,
            )
        pl.semaphore_wait(barrier_sem, STEPS)

        hidden = jnp.maximum(
            jnp.dot(x_ref[...], w1_ref[...], preferred_element_type=jnp.float32),
            0.0,
        )
        partial = jnp.dot(hidden, w2_ref[...], preferred_element_type=jnp.float32)
        acc = [partial[c * mc:(c + 1) * mc, :] for c in range(NCHUNKS)]

        def make_rdma(c, s):
            return pltpu.make_async_remote_copy(
                src_ref=comm_ref.at[c, 2 * s],
                dst_ref=comm_ref.at[c, 2 * s + 1],
                send_sem=send_sems.at[c, s],
                recv_sem=recv_sems.at[c, s],
                device_id=(partner[c][s],),
                device_id_type=pl.DeviceIdType.MESH,
            )

        rdmas = [None] * NCHUNKS
        for c in range(NCHUNKS):
            comm_ref[c, 0] = acc[c]
            rdmas[c] = make_rdma(c, 0)
            rdmas[c].start()
        for s in range(STEPS):
            for c in range(NCHUNKS):
                rdmas[c].wait()
                acc[c] = acc[c] + comm_ref[c, 2 * s + 1]
                if s + 1 < STEPS:
                    comm_ref[c, 2 * (s + 1)] = acc[c]
                    rdmas[c] = make_rdma(c, s + 1)
                    rdmas[c].start()
        for c in range(NCHUNKS):
            out_ref[c * mc:(c + 1) * mc, :] = acc[c]

    return pl.pallas_call(
        body,
        out_shape=jax.ShapeDtypeStruct((m, n), jnp.float32),
        in_specs=[
            pl.BlockSpec(memory_space=pltpu.VMEM),
            pl.BlockSpec(memory_space=pltpu.VMEM),
            pl.BlockSpec(memory_space=pltpu.VMEM),
        ],
        out_specs=pl.BlockSpec(memory_space=pltpu.VMEM),
        scratch_shapes=[
            pltpu.VMEM((NCHUNKS, 2 * STEPS, mc, n), jnp.float32),
            pltpu.SemaphoreType.DMA((NCHUNKS, STEPS)),
            pltpu.SemaphoreType.DMA((NCHUNKS, STEPS)),
        ],
        compiler_params=pltpu.CompilerParams(collective_id=0),
    )(x, W1, W2)


# device time: 18257 ns/iter; 4.2388x vs baseline; 1.0533x over previous
import jax
import jax.numpy as jnp
from jax import lax
from jax.experimental import pallas as pl
from jax.experimental.pallas import tpu as pltpu

N_DEV = 16


def kernel(x, W1, W2):
    m, _ = x.shape
    n = W2.shape[1]

    STEPS = 4
    NCHUNKS = 4
    mc = m // NCHUNKS

    def body(x_ref, w1_ref, w2_ref, out_ref, comm_ref, send_sems, recv_sems):
        my = lax.axis_index("i")
        p_in_plane = my & 3
        dim_partner = [
            my ^ 1,
            my + 3 - 2 * p_in_plane,
            my ^ 4,
            my ^ 8,
        ]
        partner = [
            [dim_partner[(s + c) % STEPS] for s in range(STEPS)]
            for c in range(NCHUNKS)
        ]
        partners_all = dim_partner

        barrier_sem = pltpu.get_barrier_semaphore()
        for p in partners_all:
            pl.semaphore_signal(
                barrier_sem, inc=1,
                device_id=(p,), device_id_type=pl.DeviceIdType.MESH,
            )
        pl.semaphore_wait(barrier_sem, STEPS)

        def make_rdma(c, s):
            return pltpu.make_async_remote_copy(
                src_ref=comm_ref.at[c, 2 * s],
                dst_ref=comm_ref.at[c, 2 * s + 1],
                send_sem=send_sems.at[c, s],
                recv_sem=recv_sems.at[c, s],
                device_id=(partner[c][s],),
                device_id_type=pl.DeviceIdType.MESH,
            )

        acc = [None] * NCHUNKS
        rdmas = [None] * NCHUNKS
        for c in range(NCHUNKS):
            hidden_c = jnp.maximum(
                jnp.dot(
                    x_ref[c * mc:(c + 1) * mc, :], w1_ref[...],
                    preferred_element_type=jnp.float32,
                ),
                0.0,
            )
            acc[c] = jnp.dot(
                hidden_c, w2_ref[...], preferred_element_type=jnp.float32
            )
            comm_ref[c, 0] = acc[c].astype(jnp.bfloat16)
            rdmas[c] = make_rdma(c, 0)
            rdmas[c].start()
        for s in range(STEPS):
            for c in range(NCHUNKS):
                rdmas[c].wait()
                acc[c] = acc[c] + comm_ref[c, 2 * s + 1][...].astype(jnp.float32)
                if s + 1 < STEPS:
                    comm_ref[c, 2 * (s + 1)] = acc[c].astype(jnp.bfloat16)
                    rdmas[c] = make_rdma(c, s + 1)
                    rdmas[c].start()
        for c in range(NCHUNKS):
            out_ref[c * mc:(c + 1) * mc, :] = acc[c]

    return pl.pallas_call(
        body,
        out_shape=jax.ShapeDtypeStruct((m, n), jnp.float32),
        in_specs=[
            pl.BlockSpec(memory_space=pltpu.VMEM),
            pl.BlockSpec(memory_space=pltpu.VMEM),
            pl.BlockSpec(memory_space=pltpu.VMEM),
        ],
        out_specs=pl.BlockSpec(memory_space=pltpu.VMEM),
        scratch_shapes=[
            pltpu.VMEM((NCHUNKS, 2 * STEPS, mc, n), jnp.bfloat16),
            pltpu.SemaphoreType.DMA((NCHUNKS, STEPS)),
            pltpu.SemaphoreType.DMA((NCHUNKS, STEPS)),
        ],
        compiler_params=pltpu.CompilerParams(collective_id=0),
    )(x, W1, W2)


# device time: 17986 ns/iter; 4.3026x vs baseline; 1.0151x over previous
import jax
import jax.numpy as jnp
from jax import lax
from jax.experimental import pallas as pl
from jax.experimental.pallas import tpu as pltpu

N_DEV = 16


def kernel(x, W1, W2):
    m, _ = x.shape
    n = W2.shape[1]

    STEPS = 4
    NCHUNKS = 4
    mc = m // NCHUNKS

    def body(x_ref, w1_ref, w2_ref, out_ref, comm_ref, send_sems, recv_sems):
        my = lax.axis_index("i")
        p_in_plane = my & 3
        dim_partner = [
            my ^ 1,
            my + 3 - 2 * p_in_plane,
            my ^ 4,
            my ^ 8,
        ]
        partner = [
            [dim_partner[(s + c) % STEPS] for s in range(STEPS)]
            for c in range(NCHUNKS)
        ]
        partners_all = dim_partner

        barrier_sem = pltpu.get_barrier_semaphore()
        for p in partners_all:
            pl.semaphore_signal(
                barrier_sem, inc=1,
                device_id=(p,), device_id_type=pl.DeviceIdType.MESH,
            )

        def make_rdma(c, s):
            return pltpu.make_async_remote_copy(
                src_ref=comm_ref.at[c, 2 * s],
                dst_ref=comm_ref.at[c, 2 * s + 1],
                send_sem=send_sems.at[c, s],
                recv_sem=recv_sems.at[c, s],
                device_id=(partner[c][s],),
                device_id_type=pl.DeviceIdType.MESH,
            )

        acc = [None] * NCHUNKS
        rdmas = [None] * NCHUNKS
        w1_b = w1_ref[...].astype(jnp.bfloat16)
        w2_b = w2_ref[...].astype(jnp.bfloat16)
        for c in range(NCHUNKS):
            hidden_c = jnp.maximum(
                jnp.dot(
                    x_ref[c * mc:(c + 1) * mc, :].astype(jnp.bfloat16), w1_b,
                    preferred_element_type=jnp.float32,
                ),
                0.0,
            )
            acc[c] = jnp.dot(
                hidden_c.astype(jnp.bfloat16), w2_b,
                preferred_element_type=jnp.float32,
            )
            comm_ref[c, 0] = acc[c].astype(jnp.bfloat16)
            if c == 0:
                pl.semaphore_wait(barrier_sem, STEPS)
            rdmas[c] = make_rdma(c, 0)
            rdmas[c].start()
        for s in range(STEPS):
            for c in range(NCHUNKS):
                rdmas[c].wait()
                add = acc[c] + comm_ref[c, 2 * s + 1][...].astype(jnp.float32)
                if s + 1 < STEPS:
                    acc[c] = add
                    comm_ref[c, 2 * (s + 1)] = add.astype(jnp.bfloat16)
                    rdmas[c] = make_rdma(c, s + 1)
                    rdmas[c].start()
                else:
                    out_ref[c * mc:(c + 1) * mc, :] = add

    return pl.pallas_call(
        body,
        out_shape=jax.ShapeDtypeStruct((m, n), jnp.float32),
        in_specs=[
            pl.BlockSpec(memory_space=pltpu.VMEM),
            pl.BlockSpec(memory_space=pltpu.VMEM),
            pl.BlockSpec(memory_space=pltpu.VMEM),
        ],
        out_specs=pl.BlockSpec(memory_space=pltpu.VMEM),
        scratch_shapes=[
            pltpu.VMEM((NCHUNKS, 2 * STEPS, mc, n), jnp.bfloat16),
            pltpu.SemaphoreType.DMA((NCHUNKS, STEPS)),
            pltpu.SemaphoreType.DMA((NCHUNKS, STEPS)),
        ],
        compiler_params=pltpu.CompilerParams(collective_id=0),
    )(x, W1, W2)
